# baseline (speedup 1.0000x reference)
import functools

import jax
import jax.numpy as jnp
from jax import lax
from jax.experimental import pallas as pl
from jax.experimental.pallas import tpu as pltpu
from jax.experimental.pallas import tpu_sc as plsc

_V = 40
_D = 128
_CHUNK = 128
_NB = 5
_NW = 32
_STEPS = 100
_Q = _STEPS // _NB
_BPW = _STEPS * _CHUNK
_NSC = _NW * _BPW
_TCBLK = 2048


def _make_sc_gather():
    mesh = plsc.VectorSubcoreMesh(core_axis_name="c", subcore_axis_name="s")

    @functools.partial(
        pl.kernel,
        mesh=mesh,
        out_type=jax.ShapeDtypeStruct((_NW, _BPW, _D), jnp.float32),
        scratch_types=[
            pltpu.VMEM_SHARED((_V, _D), jnp.float32),
            pltpu.VMEM((_STEPS, _CHUNK), jnp.int32),
            pltpu.VMEM((_NB, _CHUNK, _D), jnp.float32),
            pltpu.SemaphoreType.DMA,
        ] + [pltpu.SemaphoreType.DMA] * _NB,
    )
    def sc_gather(table_hbm, idx_hbm, out_hbm, tbl_s, idx_v, rows_v, g_sem,
                  *w_sems):
        sid = lax.axis_index("s")
        wid = sid * 2 + lax.axis_index("c")

        @pl.when(sid == 0)
        def _():
            pltpu.sync_copy(table_hbm, tbl_s)

        pltpu.sync_copy(idx_hbm.at[wid], idx_v)
        plsc.subcore_barrier()

        def issue_gather(j, u):
            pltpu.async_copy(tbl_s.at[idx_v.at[j]], rows_v.at[u], g_sem)

        def drain_gather(u):
            pltpu.make_async_copy(
                tbl_s.at[idx_v.at[0]], rows_v.at[u], g_sem).wait()

        def issue_write(j, u):
            pltpu.async_copy(
                rows_v.at[u], out_hbm.at[wid, pl.ds(j * _CHUNK, _CHUNK)],
                w_sems[u])

        def drain_write(u):
            pltpu.make_async_copy(
                rows_v.at[u], out_hbm.at[wid, pl.ds(0, _CHUNK)],
                w_sems[u]).wait()

        issue_gather(0, 0)
        issue_gather(1, 1)

        def qstep(q, carry):
            for u in range(_NB):
                j = _NB * q + u
                drain_gather(u)
                issue_write(j, u)
                un = (u + 2) % _NB

                @pl.when(j >= 3)
                def _():
                    drain_write(un)

                @pl.when(j < _STEPS - 2)
                def _():
                    issue_gather(j + 2, un)

            return carry

        lax.fori_loop(0, _Q, qstep, None)
        for u in ((_STEPS - 3) % _NB, (_STEPS - 2) % _NB, (_STEPS - 1) % _NB):
            drain_write(u)

    return sc_gather


_sc_gather = _make_sc_gather()


def _tc_body(idx_ref, tbl_ref, out_ref):
    onehot = jnp.where(
        idx_ref[...] == lax.broadcasted_iota(jnp.int32, (_TCBLK, _D), 1),
        1.0, 0.0).astype(jnp.float32)
    out_ref[...] = jnp.dot(
        onehot, tbl_ref[...], preferred_element_type=jnp.float32)


def _tc_gather(idx2, table_pad, n_rows):
    grid = n_rows // _TCBLK
    return pl.pallas_call(
        _tc_body,
        grid=(grid,),
        in_specs=[
            pl.BlockSpec((_TCBLK, 1), lambda i: (i, 0)),
            pl.BlockSpec((_D, _D), lambda i: (0, 0)),
        ],
        out_specs=pl.BlockSpec((_TCBLK, _D), lambda i: (i, 0)),
        out_shape=jax.ShapeDtypeStruct((n_rows, _D), jnp.float32),
    )(idx2, table_pad)


def kernel(indices, embeddings):
    b, l = indices.shape
    flat = indices.reshape(-1)
    idx_sc = flat[:_NSC].reshape(_NW, _STEPS, _CHUNK)
    idx_tc = flat[_NSC:].reshape(-1, 1)
    table_pad = jnp.zeros((_D, _D), jnp.float32).at[:_V].set(embeddings)

    out_sc = _sc_gather(embeddings, idx_sc).reshape(_NSC, _D)
    out_tc = _tc_gather(idx_tc, table_pad, flat.shape[0] - _NSC)
    return jnp.concatenate([out_sc, out_tc], axis=0).reshape(b, l, _D)


# speedup vs baseline: 3.5631x; 3.5631x over previous
import functools

import jax
import jax.numpy as jnp
from jax import lax
from jax.experimental import pallas as pl
from jax.experimental.pallas import tpu as pltpu
from jax.experimental.pallas import tpu_sc as plsc

_V = 40
_D = 128
_CHUNK = 128
_NB = 5
_NW = 32
_STEPS = 200
_Q = _STEPS // _NB
_BPW = _STEPS * _CHUNK


def _make_sc_gather():
    mesh = plsc.VectorSubcoreMesh(core_axis_name="c", subcore_axis_name="s")

    @functools.partial(
        pl.kernel,
        mesh=mesh,
        out_type=jax.ShapeDtypeStruct((_NW, _BPW, _D), jnp.float32),
        scratch_types=[
            pltpu.VMEM_SHARED((_V, _D), jnp.float32),
            pltpu.VMEM((_STEPS, _CHUNK), jnp.int32),
            pltpu.VMEM((_NB, _CHUNK, _D), jnp.float32),
            pltpu.SemaphoreType.DMA,
        ] + [pltpu.SemaphoreType.DMA] * _NB,
    )
    def sc_gather(table_hbm, idx_hbm, out_hbm, tbl_s, idx_v, rows_v, g_sem,
                  *w_sems):
        sid = lax.axis_index("s")
        wid = sid * 2 + lax.axis_index("c")

        @pl.when(sid == 0)
        def _():
            pltpu.sync_copy(table_hbm, tbl_s)

        pltpu.sync_copy(idx_hbm.at[wid], idx_v)
        plsc.subcore_barrier()

        def issue_gather(j, u):
            pltpu.async_copy(tbl_s.at[idx_v.at[j]], rows_v.at[u], g_sem)

        def drain_gather(u):
            pltpu.make_async_copy(
                tbl_s.at[idx_v.at[0]], rows_v.at[u], g_sem).wait()

        def issue_write(j, u):
            pltpu.async_copy(
                rows_v.at[u], out_hbm.at[wid, pl.ds(j * _CHUNK, _CHUNK)],
                w_sems[u])

        def drain_write(u):
            pltpu.make_async_copy(
                rows_v.at[u], out_hbm.at[wid, pl.ds(0, _CHUNK)],
                w_sems[u]).wait()

        issue_gather(0, 0)
        issue_gather(1, 1)

        def qstep(q, carry):
            for u in range(_NB):
                j = _NB * q + u
                drain_gather(u)
                issue_write(j, u)
                un = (u + 2) % _NB

                @pl.when(j >= 3)
                def _():
                    drain_write(un)

                @pl.when(j < _STEPS - 2)
                def _():
                    issue_gather(j + 2, un)

            return carry

        lax.fori_loop(0, _Q, qstep, None)
        for u in ((_STEPS - 3) % _NB, (_STEPS - 2) % _NB, (_STEPS - 1) % _NB):
            drain_write(u)

    return sc_gather


_sc_gather = _make_sc_gather()


def kernel(indices, embeddings):
    b, l = indices.shape
    idx = indices.reshape(_NW, _STEPS, _CHUNK)
    out = _sc_gather(embeddings, idx)
    return out.reshape(b, l, _D)
